# Initial kernel scaffold; baseline (speedup 1.0000x reference)
import jax
import jax.numpy as jnp
from jax.experimental import pallas as pl


def kernel(indices, table):
    raise NotImplementedError("write your pallas kernel here")


# speedup vs baseline: 4.9035x; 4.9035x over previous
import functools

import jax
import jax.numpy as jnp
from jax import lax
from jax.experimental import pallas as pl
from jax.experimental.pallas import tpu as pltpu
from jax.experimental.pallas import tpu_sc as plsc

NUM_EMB = 10
EMB_DIM = 5
BATCH = 16384
HIST = 200

N = BATCH * HIST
NC = 2
NS = 16
NW = NC * NS
PER_W = N // NW
CH = 5120
NCHUNK = PER_W // CH
LANES = 16


def _sc_body(idx_hbm, tab_hbm, out_hbm, tab_v, idx_v, out_v, sem_in, sem_out):
    cid = lax.axis_index("c")
    sid = lax.axis_index("s")
    wid = sid * NC + cid
    base = wid * PER_W

    pltpu.sync_copy(tab_hbm, tab_v)
    for j in range(4):
        sl = pl.ds(j * LANES, LANES)
        tab_v[sl] = jnp.maximum(tab_v[sl], 0.0)

    lane = lax.iota(jnp.int32, LANES)
    lane5 = lane * 5

    def chunk_body(c, carry):
        cbase = base + c * CH
        pltpu.sync_copy(idx_hbm.at[pl.ds(cbase, CH)], idx_v)

        def vec_body(i, carry2):
            iv = idx_v[pl.ds(i * LANES, LANES)]
            iv5 = iv * 5
            pos0 = lane5 + i * (LANES * 5)
            for d in range(EMB_DIM):
                val = plsc.load_gather(tab_v, [iv5 + d])
                plsc.store_scatter(out_v, [pos0 + d], val)
            return carry2

        lax.fori_loop(0, CH // LANES, vec_body, 0)
        pltpu.sync_copy(out_v, out_hbm.at[pl.ds(cbase * EMB_DIM, CH * EMB_DIM)])
        return carry

    lax.fori_loop(0, NCHUNK, chunk_body, 0)


@jax.jit
def _lookup(idx_flat, tab_flat):
    mesh = plsc.VectorSubcoreMesh(core_axis_name="c", subcore_axis_name="s")
    f = functools.partial(
        pl.kernel,
        mesh=mesh,
        out_type=jax.ShapeDtypeStruct((N * EMB_DIM,), jnp.float32),
        scratch_types=[
            pltpu.VMEM((64,), jnp.float32),
            pltpu.VMEM((CH,), jnp.int32),
            pltpu.VMEM((CH * EMB_DIM,), jnp.float32),
            pltpu.SemaphoreType.DMA,
            pltpu.SemaphoreType.DMA,
        ],
        compiler_params=pltpu.CompilerParams(needs_layout_passes=False),
    )(_sc_body)
    return f(idx_flat, tab_flat)


def kernel(indices, table):
    idx_flat = indices.reshape(-1).astype(jnp.int32)
    tab_flat = jnp.pad(table.reshape(-1), (0, 64 - NUM_EMB * EMB_DIM))
    out = _lookup(idx_flat, tab_flat)
    return out.reshape(BATCH, HIST, EMB_DIM)


# speedup vs baseline: 5.2317x; 1.0669x over previous
import functools

import jax
import jax.numpy as jnp
from jax import lax
from jax.experimental import pallas as pl
from jax.experimental.pallas import tpu as pltpu
from jax.experimental.pallas import tpu_sc as plsc

NUM_EMB = 10
EMB_DIM = 5
BATCH = 16384
HIST = 200

N = BATCH * HIST
NC = 2
NS = 16
NW = NC * NS
PER_W = N // NW
CH = 5120
NCHUNK = PER_W // CH
LANES = 16
UNROLL = 8


def _sc_body(idx_hbm, tab_hbm, out_hbm, tab_v,
             idx_v0, idx_v1, out_v0, out_v1,
             sem_i0, sem_i1, sem_o0, sem_o1):
    cid = lax.axis_index("c")
    sid = lax.axis_index("s")
    wid = sid * NC + cid
    base = wid * PER_W

    pltpu.sync_copy(tab_hbm, tab_v)
    for j in range(4):
        sl = pl.ds(j * LANES, LANES)
        tab_v[sl] = jnp.maximum(tab_v[sl], 0.0)

    lane5 = lax.iota(jnp.int32, LANES) * 5

    idx_bufs = (idx_v0, idx_v1)
    out_bufs = (out_v0, out_v1)
    sem_in = (sem_i0, sem_i1)
    sem_out = (sem_o0, sem_o1)

    def start_in(c):
        b = c % 2
        return pltpu.async_copy(
            idx_hbm.at[pl.ds(base + c * CH, CH)], idx_bufs[b], sem_in[b])

    def start_out(c):
        b = c % 2
        return pltpu.async_copy(
            out_bufs[b], out_hbm.at[pl.ds((base + c * CH) * EMB_DIM, CH * EMB_DIM)],
            sem_out[b])

    def compute(idx_b, out_b):
        @plsc.parallel_loop(0, CH // LANES, unroll=UNROLL)
        def body(i):
            iv5 = idx_b[pl.ds(i * LANES, LANES)] * 5
            pos0 = lane5 + i * (LANES * EMB_DIM)
            for d in range(EMB_DIM):
                val = plsc.load_gather(tab_v, [iv5 + d])
                plsc.store_scatter(out_b, [pos0 + d], val)

    in_copies = {0: start_in(0)}
    out_copies = {}
    for c in range(NCHUNK):
        b = c % 2
        in_copies[c].wait()
        if c + 1 < NCHUNK:
            in_copies[c + 1] = start_in(c + 1)
        if c >= 2:
            out_copies[c - 2].wait()
        compute(idx_bufs[b], out_bufs[b])
        out_copies[c] = start_out(c)
    out_copies[NCHUNK - 2].wait()
    out_copies[NCHUNK - 1].wait()


@jax.jit
def _lookup(idx_flat, tab_flat):
    mesh = plsc.VectorSubcoreMesh(core_axis_name="c", subcore_axis_name="s")
    f = functools.partial(
        pl.kernel,
        mesh=mesh,
        out_type=jax.ShapeDtypeStruct((N * EMB_DIM,), jnp.float32),
        scratch_types=[
            pltpu.VMEM((64,), jnp.float32),
            pltpu.VMEM((CH,), jnp.int32),
            pltpu.VMEM((CH,), jnp.int32),
            pltpu.VMEM((CH * EMB_DIM,), jnp.float32),
            pltpu.VMEM((CH * EMB_DIM,), jnp.float32),
            pltpu.SemaphoreType.DMA,
            pltpu.SemaphoreType.DMA,
            pltpu.SemaphoreType.DMA,
            pltpu.SemaphoreType.DMA,
        ],
        compiler_params=pltpu.CompilerParams(needs_layout_passes=False),
    )(_sc_body)
    return f(idx_flat, tab_flat)


def kernel(indices, table):
    idx_flat = indices.reshape(-1).astype(jnp.int32)
    tab_flat = jnp.pad(table.reshape(-1), (0, 64 - NUM_EMB * EMB_DIM))
    out = _lookup(idx_flat, tab_flat)
    return out.reshape(BATCH, HIST, EMB_DIM)


# speedup vs baseline: 5.2789x; 1.0090x over previous
import functools

import jax
import jax.numpy as jnp
from jax import lax
from jax.experimental import pallas as pl
from jax.experimental.pallas import tpu as pltpu
from jax.experimental.pallas import tpu_sc as plsc

NUM_EMB = 10
EMB_DIM = 5
BATCH = 16384
HIST = 200

NC = 2
NS = 16
NW = NC * NS
ROWS_W = BATCH // NW
RCH = 32
NCHUNK = ROWS_W // RCH
CH = RCH * HIST
LANES = 16
UNROLL = 8


def _sc_body(idx_hbm, tab_hbm, out_hbm, tab_v,
             idx_v0, idx_v1, out_v0, out_v1,
             sem_i0, sem_i1, sem_o0, sem_o1):
    cid = lax.axis_index("c")
    sid = lax.axis_index("s")
    wid = sid * NC + cid
    row0 = wid * ROWS_W

    pltpu.sync_copy(tab_hbm, tab_v)
    for j in range(4):
        sl = pl.ds(j * LANES, LANES)
        tab_v[sl] = jnp.maximum(tab_v[sl], 0.0)

    lane = lax.iota(jnp.int32, LANES)
    lane5 = lane * 5

    idx_bufs = (idx_v0, idx_v1)
    out_bufs = (out_v0, out_v1)
    sem_in = (sem_i0, sem_i1)
    sem_out = (sem_o0, sem_o1)

    def start_in(c):
        b = c % 2
        return pltpu.async_copy(
            idx_hbm.at[pl.ds(row0 + c * RCH, RCH)], idx_bufs[b], sem_in[b])

    def start_out(c):
        b = c % 2
        return pltpu.async_copy(
            out_bufs[b],
            out_hbm.at[pl.ds((row0 + c * RCH) * HIST * EMB_DIM, CH * EMB_DIM)],
            sem_out[b])

    def compute(idx_b, out_b):
        @plsc.parallel_loop(0, CH // LANES, unroll=UNROLL)
        def body(i):
            g = lane + i * LANES
            r = (g * 5243) >> 20
            col = g - r * HIST
            iv5 = plsc.load_gather(idx_b, [r, col]) * 5
            pos0 = lane5 + i * (LANES * EMB_DIM)
            for d in range(EMB_DIM):
                val = plsc.load_gather(tab_v, [iv5 + d])
                plsc.store_scatter(out_b, [pos0 + d], val)

    in_copies = {0: start_in(0)}
    out_copies = {}
    for c in range(NCHUNK):
        b = c % 2
        in_copies[c].wait()
        if c + 1 < NCHUNK:
            in_copies[c + 1] = start_in(c + 1)
        if c >= 2:
            out_copies[c - 2].wait()
        compute(idx_bufs[b], out_bufs[b])
        out_copies[c] = start_out(c)
    out_copies[NCHUNK - 2].wait()
    out_copies[NCHUNK - 1].wait()


@jax.jit
def _lookup(idx, tab_flat):
    mesh = plsc.VectorSubcoreMesh(core_axis_name="c", subcore_axis_name="s")
    f = functools.partial(
        pl.kernel,
        mesh=mesh,
        out_type=jax.ShapeDtypeStruct((BATCH * HIST * EMB_DIM,), jnp.float32),
        scratch_types=[
            pltpu.VMEM((64,), jnp.float32),
            pltpu.VMEM((RCH, HIST), jnp.int32),
            pltpu.VMEM((RCH, HIST), jnp.int32),
            pltpu.VMEM((CH * EMB_DIM,), jnp.float32),
            pltpu.VMEM((CH * EMB_DIM,), jnp.float32),
            pltpu.SemaphoreType.DMA,
            pltpu.SemaphoreType.DMA,
            pltpu.SemaphoreType.DMA,
            pltpu.SemaphoreType.DMA,
        ],
        compiler_params=pltpu.CompilerParams(needs_layout_passes=False),
    )(_sc_body)
    return f(idx, tab_flat)


def kernel(indices, table):
    tab_flat = jnp.pad(table.reshape(-1), (0, 64 - NUM_EMB * EMB_DIM))
    out = _lookup(indices.astype(jnp.int32), tab_flat)
    return out.reshape(BATCH, HIST, EMB_DIM)


# speedup vs baseline: 9.0129x; 1.7073x over previous
import functools

import jax
import jax.numpy as jnp
from jax import lax
from jax.experimental import pallas as pl
from jax.experimental.pallas import tpu as pltpu
from jax.experimental.pallas import tpu_sc as plsc

NUM_EMB = 10
EMB_DIM = 5
BATCH = 16384
HIST = 200

NC = 2
NS = 16
NW = NC * NS
ROWS_W = BATCH // NW
RCH = 8
NCHUNK = ROWS_W // RCH
NOB = 4
LANES = 16
NVEC = 13


def _sc_body(idx_hbm, tab_hbm, out_hbm, tab_v,
             idx_v0, idx_v1, ob0, ob1, ob2, ob3,
             sem_i0, sem_i1, so0, so1, so2, so3):
    cid = lax.axis_index("c")
    sid = lax.axis_index("s")
    wid = sid * NC + cid
    row0 = wid * ROWS_W

    pltpu.sync_copy(tab_hbm, tab_v)
    for j in range(4):
        sl = pl.ds(j * LANES, LANES)
        tab_v[sl] = jnp.maximum(tab_v[sl], 0.0)

    lane = lax.iota(jnp.int32, LANES)

    in_bufs = (idx_v0, idx_v1)
    sem_in = (sem_i0, sem_i1)
    out_bufs = (ob0, ob1, ob2, ob3)
    sem_out = (so0, so1, so2, so3)

    def start_in(c, b):
        return pltpu.async_copy(
            idx_hbm.at[pl.ds(row0 + c * RCH, RCH)], in_bufs[b], sem_in[b])

    def compute_row(in_buf, rl, ob):
        rvec = jnp.full((LANES,), rl, jnp.int32)
        for i in range(NVEC):
            c0 = i * LANES if i < NVEC - 1 else HIST - LANES
            col = lane + c0
            iv5 = plsc.load_gather(in_buf, [rvec, col]) * 5
            for d in range(EMB_DIM):
                val = plsc.load_gather(tab_v, [iv5 + d])
                dvec = jnp.full((LANES,), d, jnp.int32)
                plsc.store_scatter(ob, [col, dvec], val)

    start_in(0, 0)
    start_in(1, 1)

    @pl.loop(0, NCHUNK, step=2)
    def _chunks(g):
        for sub in range(2):
            c = g + sub
            pltpu.make_async_copy(
                idx_hbm.at[pl.ds(row0 + c * RCH, RCH)],
                in_bufs[sub], sem_in[sub]).wait()
            for rl in range(RCH):
                ob = out_bufs[rl % NOB]
                so = sem_out[rl % NOB]
                gr = c * RCH + rl

                @pl.when(gr >= NOB)
                def _drain():
                    pltpu.make_async_copy(ob, out_hbm.at[row0 + gr], so).wait()

                compute_row(in_bufs[sub], rl, ob)
                pltpu.async_copy(ob, out_hbm.at[row0 + gr], so)

            @pl.when(c + 2 < NCHUNK)
            def _next_in():
                start_in(c + 2, sub)

    for b in range(NOB):
        pltpu.make_async_copy(
            out_bufs[b], out_hbm.at[row0], sem_out[b]).wait()


@jax.jit
def _lookup(idx, tab_flat):
    mesh = plsc.VectorSubcoreMesh(core_axis_name="c", subcore_axis_name="s")
    f = functools.partial(
        pl.kernel,
        mesh=mesh,
        out_type=jax.ShapeDtypeStruct((BATCH, HIST, EMB_DIM), jnp.float32),
        scratch_types=[
            pltpu.VMEM((64,), jnp.float32),
            pltpu.VMEM((RCH, HIST), jnp.int32),
            pltpu.VMEM((RCH, HIST), jnp.int32),
            pltpu.VMEM((HIST, EMB_DIM), jnp.float32),
            pltpu.VMEM((HIST, EMB_DIM), jnp.float32),
            pltpu.VMEM((HIST, EMB_DIM), jnp.float32),
            pltpu.VMEM((HIST, EMB_DIM), jnp.float32),
            pltpu.SemaphoreType.DMA,
            pltpu.SemaphoreType.DMA,
            pltpu.SemaphoreType.DMA,
            pltpu.SemaphoreType.DMA,
            pltpu.SemaphoreType.DMA,
            pltpu.SemaphoreType.DMA,
        ],
        compiler_params=pltpu.CompilerParams(needs_layout_passes=False),
    )(_sc_body)
    return f(idx, tab_flat)


def kernel(indices, table):
    tab_flat = jnp.pad(table.reshape(-1), (0, 64 - NUM_EMB * EMB_DIM))
    return _lookup(indices.astype(jnp.int32), tab_flat)


# speedup vs baseline: 214.6873x; 23.8199x over previous
import functools

import jax
import jax.numpy as jnp
from jax import lax
from jax.experimental import pallas as pl
from jax.experimental.pallas import tpu as pltpu
from jax.experimental.pallas import tpu_sc as plsc

NUM_EMB = 10
EMB_DIM = 5
BATCH = 16384
HIST = 200

NC = 2
NS = 16
NW = NC * NS
BW = BATCH // NW
LCH = 8
NCHUNK = HIST // LCH
LANES = 16
NV = BW // LANES
UNROLL = 8


def _sc_body(idx_hbm, tab_hbm, out_hbm, tab_v,
             idx_v0, idx_v1, out_v0, out_v1,
             sem_i0, sem_i1, sem_o0, sem_o1):
    cid = lax.axis_index("c")
    sid = lax.axis_index("s")
    wid = sid * NC + cid
    b0 = wid * BW

    pltpu.sync_copy(tab_hbm, tab_v)
    for j in range(4):
        sl = pl.ds(j * LANES, LANES)
        tab_v[sl] = jnp.maximum(tab_v[sl], 0.0)

    in_bufs = (idx_v0, idx_v1)
    out_bufs = (out_v0, out_v1)
    sem_in = (sem_i0, sem_i1)
    sem_out = (sem_o0, sem_o1)

    def start_in(c):
        b = c % 2
        return pltpu.async_copy(
            idx_hbm.at[pl.ds(c * LCH, LCH), pl.ds(b0, BW)],
            in_bufs[b], sem_in[b])

    def start_out(c):
        b = c % 2
        return pltpu.async_copy(
            out_bufs[b],
            out_hbm.at[:, pl.ds(c * LCH, LCH), pl.ds(b0, BW)],
            sem_out[b])

    def compute(in_b, out_b):
        @plsc.parallel_loop(0, LCH * NV, unroll=UNROLL)
        def body(i):
            l = i >> 5
            v = (i & (NV - 1)) * LANES
            iv5 = in_b[l, pl.ds(v, LANES)] * 5
            for d in range(EMB_DIM):
                out_b[d, l, pl.ds(v, LANES)] = plsc.load_gather(
                    tab_v, [iv5 + d])

    in_copies = {0: start_in(0)}
    out_copies = {}
    for c in range(NCHUNK):
        b = c % 2
        in_copies[c].wait()
        if c + 1 < NCHUNK:
            in_copies[c + 1] = start_in(c + 1)
        if c >= 2:
            out_copies[c - 2].wait()
        compute(in_bufs[b], out_bufs[b])
        out_copies[c] = start_out(c)
    out_copies[NCHUNK - 2].wait()
    out_copies[NCHUNK - 1].wait()


@jax.jit
def _lookup(idx_t, tab_flat):
    mesh = plsc.VectorSubcoreMesh(core_axis_name="c", subcore_axis_name="s")
    f = functools.partial(
        pl.kernel,
        mesh=mesh,
        out_type=jax.ShapeDtypeStruct((EMB_DIM, HIST, BATCH), jnp.float32),
        scratch_types=[
            pltpu.VMEM((64,), jnp.float32),
            pltpu.VMEM((LCH, BW), jnp.int32),
            pltpu.VMEM((LCH, BW), jnp.int32),
            pltpu.VMEM((EMB_DIM, LCH, BW), jnp.float32),
            pltpu.VMEM((EMB_DIM, LCH, BW), jnp.float32),
            pltpu.SemaphoreType.DMA,
            pltpu.SemaphoreType.DMA,
            pltpu.SemaphoreType.DMA,
            pltpu.SemaphoreType.DMA,
        ],
        compiler_params=pltpu.CompilerParams(needs_layout_passes=False),
    )(_sc_body)
    return f(idx_t, tab_flat)


def kernel(indices, table):
    tab_flat = jnp.pad(table.reshape(-1), (0, 64 - NUM_EMB * EMB_DIM))
    idx_t = jnp.transpose(indices.astype(jnp.int32))
    out_t = _lookup(idx_t, tab_flat)
    return jnp.transpose(out_t, (2, 1, 0))
